# Initial kernel scaffold; baseline (speedup 1.0000x reference)
import jax
import jax.numpy as jnp
from jax.experimental import pallas as pl


def kernel(flat, counts, cu_seqlens, n_repeats_table, mixer_w):
    raise NotImplementedError("write your pallas kernel here")


# speedup vs baseline: 3.3752x; 3.3752x over previous
import functools

import jax
import jax.numpy as jnp
from jax import lax
from jax.experimental import pallas as pl
from jax.experimental.pallas import tpu as pltpu
from jax.experimental.pallas import tpu_sc as plsc

N = 8192
DM = 8
DE = 64
NSEG = 8
MAXP1 = 8
L = 16
NC, NS = 2, 16
SCHUNK = N // NS
CHUNK = N // (NC * NS)
SROW = 16

_mesh = plsc.VectorSubcoreMesh(
    core_axis_name="c", subcore_axis_name="s", num_cores=NC, num_subcores=NS)


def _take(v, lane):
    idx = jnp.full((L,), lane, jnp.int32) if isinstance(lane, int) else lane
    return jnp.take_along_axis(v, idx, axis=0)


def _seg_vec(tvec, bcs):
    svec = jnp.zeros((L,), jnp.int32)
    for bc in bcs:
        svec = svec + jnp.where(tvec >= bc, 1, 0).astype(jnp.int32)
    return svec


@functools.partial(
    pl.kernel,
    out_type=jax.ShapeDtypeStruct((N * DE,), jnp.float32),
    mesh=_mesh,
    compiler_params=pltpu.CompilerParams(needs_layout_passes=False),
    scratch_types=[
        pltpu.VMEM((SCHUNK * DM + 8,), jnp.float32),
        pltpu.VMEM((SCHUNK,), jnp.int32),
        pltpu.VMEM((L,), jnp.int32),
        pltpu.VMEM((MAXP1 * DE,), jnp.float32),
        pltpu.VMEM((DM * DE,), jnp.float32),
        pltpu.VMEM((NSEG * SROW,), jnp.float32),
        pltpu.VMEM_SHARED((NS * NSEG * SROW,), jnp.float32),
        pltpu.VMEM((NS * NSEG * SROW,), jnp.float32),
        pltpu.VMEM((MAXP1 * NSEG * DE,), jnp.float32),
        pltpu.VMEM((CHUNK * DE,), jnp.float32),
        pltpu.SemaphoreType.DMA,
    ],
)
def _fused_call(flat_hbm, counts_hbm, cu_hbm, table_hbm, w_hbm, out_hbm,
                flat_v, counts_v, cu_v, table_v, w_v, stats_v,
                shared_x, parts_v, t2_v, out_v, sem):
    cid = lax.axis_index("c")
    sid = lax.axis_index("s")
    sbase = sid * SCHUNK
    obase = sbase + cid * CHUNK

    cps = []
    cps.append(pltpu.async_copy(
        flat_hbm.at[pl.ds(sbase * DM, SCHUNK * DM)],
        flat_v.at[pl.ds(0, SCHUNK * DM)], sem))
    cps.append(pltpu.async_copy(
        counts_hbm.at[pl.ds(sbase, SCHUNK)], counts_v, sem))
    cps.append(pltpu.async_copy(cu_hbm, cu_v.at[pl.ds(0, 9)], sem))
    cps.append(pltpu.async_copy(table_hbm, table_v, sem))
    cps.append(pltpu.async_copy(w_hbm, w_v, sem))
    for cp in cps:
        cp.wait()

    zero = jnp.zeros((L,), jnp.float32)
    for s in range(NSEG):
        stats_v[pl.ds(s * SROW, L)] = zero

    iota = lax.iota(jnp.int32, L)
    cu_vec = cu_v[...]
    bcs = [_take(cu_vec, b) for b in range(1, NSEG)]
    half = jnp.where(iota >= 8, 1, 0).astype(jnp.int32)
    lane7 = iota - 8 * half
    ones = jnp.ones((L,), jnp.float32)

    def sgrp(g, carry):
        tvec = jnp.full((L,), sbase + g * L, jnp.int32) + iota
        cvec = counts_v[pl.ds(g * L, L)]
        svec = _seg_vec(tvec, bcs)
        s16 = svec * SROW
        plsc.addupdate_scatter(stats_v, [s16 + (MAXP1 + cvec)], ones)
        for jj in range(8):
            fv = flat_v[pl.ds(g * L * DM + jj * L, L)]
            pat = jnp.full((L,), 2 * jj, jnp.int32) + half
            sexp = _take(svec, pat)
            plsc.addupdate_scatter(stats_v, [sexp * SROW + lane7], fv)
        return carry

    lax.fori_loop(0, SCHUNK // L, sgrp, 0)

    pltpu.sync_copy(stats_v, shared_x.at[pl.ds(sid * NSEG * SROW, NSEG * SROW)])
    plsc.subcore_barrier()
    pltpu.sync_copy(shared_x, parts_v)

    stats = []
    for s in range(NSEG):
        acc = jnp.zeros((L,), jnp.float32)
        for w in range(NS):
            acc = acc + parts_v[pl.ds(w * NSEG * SROW + s * SROW, L)]
        stats.append(acc)

    pooled_acc = [[jnp.zeros((L,), jnp.float32) for _ in range(4)]
                  for _ in range(NSEG)]
    for k in range(SROW):
        if k < DM:
            wrows = [w_v[pl.ds(k * DE + d * L, L)] for d in range(4)]
        else:
            wrows = [table_v[pl.ds((k - DM) * DE + d * L, L)] for d in range(4)]
        for s in range(NSEG):
            coef = _take(stats[s], k)
            for d in range(4):
                pooled_acc[s][d] = pooled_acc[s][d] + coef * wrows[d]

    for c in range(MAXP1):
        trows = [table_v[pl.ds(c * DE + d * L, L)] for d in range(4)]
        for s in range(NSEG):
            for d in range(4):
                t2_v[pl.ds((c * NSEG + s) * DE + d * L, L)] = (
                    trows[d] + pooled_acc[s][d])

    ebase = [jnp.full((L,), d * L, jnp.int32) + iota for d in range(4)]
    wr = [[w_v[pl.ds(k * DE + d * L, L)] for d in range(4)] for k in range(DM)]
    lbase = cid * CHUNK

    def ogrp(g, carry):
        tvec = jnp.full((L,), obase + g * L, jnp.int32) + iota
        cvec = counts_v[pl.ds(lbase + g * L, L)]
        svec = _seg_vec(tvec, bcs)
        cs64 = (cvec * NSEG + svec) * DE
        for j in range(L):
            iloc = lbase + g * L + j
            f2 = flat_v[pl.ds(iloc * DM, L)]
            csb = _take(cs64, j)
            accs = [plsc.load_gather(t2_v, [csb + ebase[d]]) for d in range(4)]
            for k in range(DM):
                fk = _take(f2, k)
                for d in range(4):
                    accs[d] = accs[d] + fk * wr[k][d]
            for d in range(4):
                out_v[pl.ds((g * L + j) * DE + d * L, L)] = accs[d]
        return carry

    lax.fori_loop(0, CHUNK // L, ogrp, 0)
    pltpu.sync_copy(out_v, out_hbm.at[pl.ds(obase * DE, CHUNK * DE)])


def kernel(flat, counts, cu_seqlens, n_repeats_table, mixer_w):
    out1 = _fused_call(
        flat.reshape(-1), counts.astype(jnp.int32),
        cu_seqlens.astype(jnp.int32),
        n_repeats_table.reshape(-1), mixer_w.reshape(-1))
    return out1.reshape(N, DE)


# speedup vs baseline: 3.4300x; 1.0162x over previous
import functools

import jax
import jax.numpy as jnp
from jax import lax
from jax.experimental import pallas as pl
from jax.experimental.pallas import tpu as pltpu
from jax.experimental.pallas import tpu_sc as plsc

N = 8192
DM = 8
DE = 64
NSEG = 8
MAXP1 = 8
L = 16
NC, NS = 2, 16
SCHUNK = N // NS
CHUNK = N // (NC * NS)
SROW = 16

_mesh = plsc.VectorSubcoreMesh(
    core_axis_name="c", subcore_axis_name="s", num_cores=NC, num_subcores=NS)


def _take(v, lane):
    idx = jnp.full((L,), lane, jnp.int32) if isinstance(lane, int) else lane
    return jnp.take_along_axis(v, idx, axis=0)


def _seg_vec(tvec, bcs):
    svec = jnp.zeros((L,), jnp.int32)
    for bc in bcs:
        svec = svec + jnp.where(tvec >= bc, 1, 0).astype(jnp.int32)
    return svec


@functools.partial(
    pl.kernel,
    out_type=jax.ShapeDtypeStruct((N, DE), jnp.float32),
    mesh=_mesh,
    compiler_params=pltpu.CompilerParams(needs_layout_passes=False),
    scratch_types=[
        pltpu.VMEM((SCHUNK, DM), jnp.float32),
        pltpu.VMEM((SCHUNK,), jnp.int32),
        pltpu.VMEM((L,), jnp.int32),
        pltpu.VMEM((MAXP1, DE), jnp.float32),
        pltpu.VMEM((DM, DE), jnp.float32),
        pltpu.VMEM((NSEG * SROW,), jnp.float32),
        pltpu.VMEM_SHARED((NS * NSEG * SROW,), jnp.float32),
        pltpu.VMEM((NS * NSEG * SROW,), jnp.float32),
        pltpu.VMEM((MAXP1 * NSEG * DE,), jnp.float32),
        pltpu.VMEM((CHUNK, DE), jnp.float32),
        pltpu.SemaphoreType.DMA,
    ],
)
def _fused_call(flat_hbm, counts_hbm, cu_hbm, table_hbm, w_hbm, out_hbm,
                flat_v, counts_v, cu_v, table_v, w_v, stats_v,
                shared_x, parts_v, t2_v, out_v, sem):
    cid = lax.axis_index("c")
    sid = lax.axis_index("s")
    sbase = sid * SCHUNK
    obase = sbase + cid * CHUNK

    cps = []
    cps.append(pltpu.async_copy(
        flat_hbm.reshape(NS, SCHUNK, DM).at[sid], flat_v, sem))
    cps.append(pltpu.async_copy(
        counts_hbm.at[pl.ds(sbase, SCHUNK)], counts_v, sem))
    cps.append(pltpu.async_copy(cu_hbm, cu_v.at[pl.ds(0, 9)], sem))
    cps.append(pltpu.async_copy(table_hbm, table_v, sem))
    cps.append(pltpu.async_copy(w_hbm, w_v, sem))
    for cp in cps:
        cp.wait()

    zero = jnp.zeros((L,), jnp.float32)
    for s in range(NSEG):
        stats_v[pl.ds(s * SROW, L)] = zero

    iota = lax.iota(jnp.int32, L)
    cu_vec = cu_v[...]
    bcs = [_take(cu_vec, b) for b in range(1, NSEG)]
    half = jnp.where(iota >= 8, 1, 0).astype(jnp.int32)
    lane7 = iota - 8 * half
    ones = jnp.ones((L,), jnp.float32)

    def sgrp(g, carry):
        tvec = jnp.full((L,), sbase + g * L, jnp.int32) + iota
        cvec = counts_v[pl.ds(g * L, L)]
        svec = _seg_vec(tvec, bcs)
        s16 = svec * SROW
        plsc.addupdate_scatter(stats_v, [s16 + (MAXP1 + cvec)], ones)
        for jj in range(8):
            rows = jnp.full((L,), g * L + 2 * jj, jnp.int32) + half
            fv = plsc.load_gather(flat_v, [rows, lane7])
            sexp = _take(svec, jnp.full((L,), 2 * jj, jnp.int32) + half)
            plsc.addupdate_scatter(stats_v, [sexp * SROW + lane7], fv)
        return carry

    lax.fori_loop(0, SCHUNK // L, sgrp, 0)

    pltpu.sync_copy(stats_v, shared_x.at[pl.ds(sid * NSEG * SROW, NSEG * SROW)])
    plsc.subcore_barrier()
    pltpu.sync_copy(shared_x, parts_v)

    stats = []
    for s in range(NSEG):
        acc = jnp.zeros((L,), jnp.float32)
        for w in range(NS):
            acc = acc + parts_v[pl.ds(w * NSEG * SROW + s * SROW, L)]
        stats.append(acc)

    pooled_acc = [[jnp.zeros((L,), jnp.float32) for _ in range(4)]
                  for _ in range(NSEG)]
    for k in range(SROW):
        if k < DM:
            wrows = [w_v[k, pl.ds(d * L, L)] for d in range(4)]
        else:
            wrows = [table_v[k - DM, pl.ds(d * L, L)] for d in range(4)]
        for s in range(NSEG):
            coef = _take(stats[s], k)
            for d in range(4):
                pooled_acc[s][d] = pooled_acc[s][d] + coef * wrows[d]

    for c in range(MAXP1):
        trows = [table_v[c, pl.ds(d * L, L)] for d in range(4)]
        for s in range(NSEG):
            for d in range(4):
                t2_v[pl.ds((c * NSEG + s) * DE + d * L, L)] = (
                    trows[d] + pooled_acc[s][d])

    ebase = [jnp.full((L,), d * L, jnp.int32) + iota for d in range(4)]
    wr = [[w_v[k, pl.ds(d * L, L)] for d in range(4)] for k in range(DM)]
    lbase = cid * CHUNK

    def ogrp(g, carry):
        tvec = jnp.full((L,), obase + g * L, jnp.int32) + iota
        cvec = counts_v[pl.ds(lbase + g * L, L)]
        svec = _seg_vec(tvec, bcs)
        cs64 = (cvec * NSEG + svec) * DE
        for j in range(L):
            iloc = lbase + g * L + j
            f2 = plsc.load_gather(flat_v, [jnp.full((L,), iloc, jnp.int32), lane7])
            csb = _take(cs64, j)
            accs = [plsc.load_gather(t2_v, [csb + ebase[d]]) for d in range(4)]
            for k in range(DM):
                fk = _take(f2, k)
                for d in range(4):
                    accs[d] = accs[d] + fk * wr[k][d]
            for d in range(4):
                out_v[g * L + j, pl.ds(d * L, L)] = accs[d]
        return carry

    lax.fori_loop(0, CHUNK // L, ogrp, 0)
    pltpu.sync_copy(out_v, out_hbm.reshape(NC * NS, CHUNK, DE).at[sid * NC + cid])


def kernel(flat, counts, cu_seqlens, n_repeats_table, mixer_w):
    return _fused_call(flat, counts, cu_seqlens, n_repeats_table, mixer_w)


# speedup vs baseline: 3.5322x; 1.0298x over previous
import functools

import jax
import jax.numpy as jnp
from jax import lax
from jax.experimental import pallas as pl
from jax.experimental.pallas import tpu as pltpu
from jax.experimental.pallas import tpu_sc as plsc

N = 8192
DM = 8
DE = 64
NSEG = 8
MAXP1 = 8
L = 16
NC, NS = 2, 16
SCHUNK = N // NS
CHUNK = N // (NC * NS)
SROW = 16

_mesh = plsc.VectorSubcoreMesh(
    core_axis_name="c", subcore_axis_name="s", num_cores=NC, num_subcores=NS)


def _take(v, lane):
    idx = jnp.full((L,), lane, jnp.int32) if isinstance(lane, int) else lane
    return jnp.take_along_axis(v, idx, axis=0)


def _seg_vec(tvec, bcs):
    svec = jnp.zeros((L,), jnp.int32)
    for bc in bcs:
        svec = svec + jnp.where(tvec >= bc, 1, 0).astype(jnp.int32)
    return svec


@functools.partial(
    pl.kernel,
    out_type=jax.ShapeDtypeStruct((N, DE), jnp.float32),
    mesh=_mesh,
    compiler_params=pltpu.CompilerParams(needs_layout_passes=False),
    scratch_types=[
        pltpu.VMEM((SCHUNK, DM), jnp.float32),
        pltpu.VMEM((SCHUNK,), jnp.int32),
        pltpu.VMEM((L,), jnp.int32),
        pltpu.VMEM((MAXP1, DE), jnp.float32),
        pltpu.VMEM((DM, DE), jnp.float32),
        pltpu.VMEM((NSEG * SROW,), jnp.float32),
        pltpu.VMEM_SHARED((NS * NSEG * SROW,), jnp.float32),
        pltpu.VMEM((NS * NSEG * SROW,), jnp.float32),
        pltpu.VMEM((MAXP1 * NSEG * DE,), jnp.float32),
        pltpu.VMEM((CHUNK, DE), jnp.float32),
        pltpu.SemaphoreType.DMA,
    ],
)
def _fused_call(flat_hbm, counts_hbm, cu_hbm, table_hbm, w_hbm, out_hbm,
                flat_v, counts_v, cu_v, table_v, w_v, stats_v,
                shared_x, parts_v, t2_v, out_v, sem):
    cid = lax.axis_index("c")
    sid = lax.axis_index("s")
    sbase = sid * SCHUNK
    obase = sbase + cid * CHUNK

    cps = []
    cps.append(pltpu.async_copy(
        flat_hbm.reshape(NS, SCHUNK, DM).at[sid], flat_v, sem))
    cps.append(pltpu.async_copy(
        counts_hbm.at[pl.ds(sbase, SCHUNK)], counts_v, sem))
    cps.append(pltpu.async_copy(cu_hbm, cu_v.at[pl.ds(0, 9)], sem))
    cps.append(pltpu.async_copy(table_hbm, table_v, sem))
    cps.append(pltpu.async_copy(w_hbm, w_v, sem))
    for cp in cps:
        cp.wait()

    zero = jnp.zeros((L,), jnp.float32)
    for s in range(NSEG):
        stats_v[pl.ds(s * SROW, L)] = zero

    iota = lax.iota(jnp.int32, L)
    cu_vec = cu_v[...]
    bcs = [_take(cu_vec, b) for b in range(1, NSEG)]
    half = jnp.where(iota >= 8, 1, 0).astype(jnp.int32)
    lane7 = iota - 8 * half
    ones = jnp.ones((L,), jnp.float32)

    def sgrp(g, carry):
        tvec = jnp.full((L,), sbase + g * L, jnp.int32) + iota
        cvec = counts_v[pl.ds(g * L, L)]
        svec = _seg_vec(tvec, bcs)
        s16 = svec * SROW
        plsc.addupdate_scatter(stats_v, [s16 + (MAXP1 + cvec)], ones)
        for jj in range(8):
            rows = jnp.full((L,), g * L + 2 * jj, jnp.int32) + half
            fv = plsc.load_gather(flat_v, [rows, lane7])
            sexp = _take(svec, jnp.full((L,), 2 * jj, jnp.int32) + half)
            plsc.addupdate_scatter(stats_v, [sexp * SROW + lane7], fv)
        return carry

    lax.fori_loop(0, SCHUNK // L, sgrp, 0)

    pltpu.sync_copy(stats_v, shared_x.at[pl.ds(sid * NSEG * SROW, NSEG * SROW)])
    plsc.subcore_barrier()
    pltpu.sync_copy(shared_x, parts_v)

    def _red(w, accs):
        return tuple(
            accs[s] + parts_v[pl.ds(w * NSEG * SROW + s * SROW, L)]
            for s in range(NSEG))

    stats = lax.fori_loop(
        0, NS, _red, tuple(jnp.zeros((L,), jnp.float32) for _ in range(NSEG)))

    def _pool_w(k, accs):
        wrows = [w_v[k, pl.ds(d * L, L)] for d in range(4)]
        kidx = jnp.full((L,), k, jnp.int32)
        return tuple(
            tuple(accs[s][d] + _take(stats[s], kidx) * wrows[d]
                  for d in range(4))
            for s in range(NSEG))

    def _pool_t(k, accs):
        wrows = [table_v[k, pl.ds(d * L, L)] for d in range(4)]
        kidx = jnp.full((L,), DM + k, jnp.int32)
        return tuple(
            tuple(accs[s][d] + _take(stats[s], kidx) * wrows[d]
                  for d in range(4))
            for s in range(NSEG))

    zero4 = tuple(tuple(jnp.zeros((L,), jnp.float32) for _ in range(4))
                  for _ in range(NSEG))
    pooled_acc = lax.fori_loop(0, DM, _pool_w, zero4)
    pooled_acc = lax.fori_loop(0, MAXP1, _pool_t, pooled_acc)

    def _t2(c, carry):
        trows = [table_v[c, pl.ds(d * L, L)] for d in range(4)]
        for s in range(NSEG):
            for d in range(4):
                t2_v[pl.ds(c * NSEG * DE + s * DE + d * L, L)] = (
                    trows[d] + pooled_acc[s][d])
        return carry

    lax.fori_loop(0, MAXP1, _t2, 0)

    ebase = [jnp.full((L,), d * L, jnp.int32) + iota for d in range(4)]
    wr = [[w_v[k, pl.ds(d * L, L)] for d in range(4)] for k in range(DM)]
    lbase = cid * CHUNK

    def ogrp(g, carry):
        tvec = jnp.full((L,), obase + g * L, jnp.int32) + iota
        cvec = counts_v[pl.ds(lbase + g * L, L)]
        svec = _seg_vec(tvec, bcs)
        cs64 = (cvec * NSEG + svec) * DE
        for j in range(L):
            iloc = lbase + g * L + j
            f2 = plsc.load_gather(
                flat_v, [jnp.full((L,), iloc, jnp.int32), lane7])
            csb = _take(cs64, j)
            accs = [plsc.load_gather(t2_v, [csb + ebase[d]]) for d in range(4)]
            for k in range(DM):
                fk = f2[k]
                for d in range(4):
                    accs[d] = accs[d] + fk * wr[k][d]
            for d in range(4):
                out_v[g * L + j, pl.ds(d * L, L)] = accs[d]
        return carry

    lax.fori_loop(0, CHUNK // L, ogrp, 0)
    pltpu.sync_copy(out_v, out_hbm.reshape(NC * NS, CHUNK, DE).at[sid * NC + cid])


def kernel(flat, counts, cu_seqlens, n_repeats_table, mixer_w):
    return _fused_call(flat, counts, cu_seqlens, n_repeats_table, mixer_w)
